# baseline (speedup 1.0000x reference)
import jax
import jax.numpy as jnp
from jax.experimental import pallas as pl

_NUM_TYPE = 1000
_BATCH = 1024
_HIST = 200
_TN = 200


def _body(ti_ref, tjs_ref, h_ref, emb_ref, out_ref):
    gtT = jnp.exp(tjs_ref[0, :][:, None] - ti_ref[:, 0][None, :])
    hf = h_ref[0].astype(jnp.float32)
    P = jax.lax.dot_general(
        emb_ref[:], hf, (((1,), (1,)), ((), ())),
        preferred_element_type=jnp.float32)
    out_ref[:] = jnp.dot(P, gtT, preferred_element_type=jnp.float32)


def kernel(ti, tjs, ci, cjs, emb_weight):
    del ci
    grid = (_NUM_TYPE // _TN,)
    out = pl.pallas_call(
        _body,
        grid=grid,
        in_specs=[
            pl.BlockSpec((_BATCH, 1), lambda i: (0, 0)),
            pl.BlockSpec((1, _HIST), lambda i: (0, 0)),
            pl.BlockSpec((1, _HIST, _NUM_TYPE), lambda i: (0, 0, 0)),
            pl.BlockSpec((_TN, _NUM_TYPE), lambda i: (i, 0)),
        ],
        out_specs=pl.BlockSpec((_TN, _BATCH), lambda i: (i, 0)),
        out_shape=jax.ShapeDtypeStruct((_NUM_TYPE, _BATCH), jnp.float32),
    )(ti, tjs, cjs, emb_weight)
    return out[:, :, None]
